# Initial kernel scaffold; baseline (speedup 1.0000x reference)
import jax
import jax.numpy as jnp
from jax.experimental import pallas as pl


def kernel(x, W_src, b_src, W_dst, b_dst, attn):
    raise NotImplementedError("write your pallas kernel here")


# speedup vs baseline: 47.0800x; 47.0800x over previous
import jax
import jax.numpy as jnp
from jax.experimental import pallas as pl

_B, _Wdim, _F = 16, 256, 64
_H, _D = 2, 256
_ALPHA = 0.2


def _gat_batch_kernel(x_ref, wt_ref, bb_ref, attn_ref, o_ref):
    xb = x_ref[0]
    p = jnp.dot(wt_ref[...], xb, preferred_element_type=jnp.float32)
    p = p + bb_ref[...]

    def head(h):
        s = p[h * _D:(h + 1) * _D]
        t = p[(2 + h) * _D:(3 + h) * _D]
        ah = attn_ref[h]
        z = s[:, :, None] + t[:, None, :]
        z = jnp.maximum(z, _ALPHA * z)
        e = jnp.sum(z * ah[:, :, None], axis=0)
        m = jnp.max(e, axis=0, keepdims=True)
        ex = jnp.exp(e - m)
        a = ex / jnp.sum(ex, axis=0, keepdims=True)
        return jnp.dot(s, a, preferred_element_type=jnp.float32)

    o_ref[0] = 0.5 * (head(0) + head(1))


def kernel(x, W_src, b_src, W_dst, b_dst, attn):
    wt = jnp.concatenate([W_src.T, W_dst.T], axis=0)
    bb = jnp.concatenate([b_src, b_dst])[:, None]
    attn_b = jnp.broadcast_to(attn[:, :, None], (_H, _D, _F))

    grid = (_B,)
    out = pl.pallas_call(
        _gat_batch_kernel,
        grid=grid,
        in_specs=[
            pl.BlockSpec((1, _Wdim, _F), lambda b: (b, 0, 0)),
            pl.BlockSpec((4 * _D, _Wdim), lambda b: (0, 0)),
            pl.BlockSpec((4 * _D, 1), lambda b: (0, 0)),
            pl.BlockSpec((_H, _D, _F), lambda b: (0, 0, 0)),
        ],
        out_specs=pl.BlockSpec((1, _D, _F), lambda b: (b, 0, 0)),
        out_shape=jax.ShapeDtypeStruct((_B, _D, _F), jnp.float32),
    )(x, wt, bb, attn_b)
    return out
